# Initial kernel scaffold; baseline (speedup 1.0000x reference)
import jax
import jax.numpy as jnp
from jax.experimental import pallas as pl


def kernel(x, embed):
    raise NotImplementedError("write your pallas kernel here")


# speedup vs baseline: 1.0027x; 1.0027x over previous
import jax
import jax.numpy as jnp
from jax.experimental import pallas as pl
from jax.experimental.pallas import tpu as pltpu

NUM_CODES = 8192
CODE_DIM = 32
TILE_N = 128


def _vq_tile_kernel(x_ref, embed_ref, dist_ref, onehot_ref, quant_ref, ind_ref):
    x = x_ref[...]
    emb = embed_ref[...]
    x2 = jnp.sum(x * x, axis=-1, keepdims=True)
    e2 = jnp.sum(emb * emb, axis=-1)
    xe = jax.lax.dot_general(
        x, emb, (((1,), (1,)), ((), ())),
        preferred_element_type=jnp.float32)
    dist = -(x2 - 2.0 * xe + e2[None, :])
    dist_ref[...] = dist
    ind = jnp.argmax(dist, axis=-1)
    iota = jax.lax.broadcasted_iota(jnp.int32, dist.shape, 1)
    onehot = (iota == ind[:, None]).astype(jnp.float32)
    onehot_ref[...] = onehot
    quant_ref[...] = jax.lax.dot_general(
        onehot, emb, (((1,), (0,)), ((), ())),
        preferred_element_type=jnp.float32,
        precision=jax.lax.Precision.HIGHEST)
    ind_ref[0, 0, :] = ind


def kernel(x, embed):
    x = x.astype(jnp.float32)
    b, t, d = x.shape
    n = b * t
    c = embed.shape[1]
    n_tiles = n // TILE_N
    xf = x.reshape(n, d)
    emb = embed.reshape(c, d)

    dist, onehot, quant, ind = pl.pallas_call(
        _vq_tile_kernel,
        grid=(n_tiles,),
        in_specs=[
            pl.BlockSpec((TILE_N, d), lambda i: (i, 0)),
            pl.BlockSpec((c, d), lambda i: (0, 0)),
        ],
        out_specs=[
            pl.BlockSpec((TILE_N, c), lambda i: (i, 0)),
            pl.BlockSpec((TILE_N, c), lambda i: (i, 0)),
            pl.BlockSpec((TILE_N, d), lambda i: (i, 0)),
            pl.BlockSpec((1, 1, TILE_N), lambda i: (i, 0, 0)),
        ],
        out_shape=[
            jax.ShapeDtypeStruct((n, c), jnp.float32),
            jax.ShapeDtypeStruct((n, c), jnp.float32),
            jax.ShapeDtypeStruct((n, d), jnp.float32),
            jax.ShapeDtypeStruct((n_tiles, 1, TILE_N), jnp.int32),
        ],
        compiler_params=pltpu.CompilerParams(
            dimension_semantics=("parallel",)),
    )(xf, emb)

    embed_ind = ind.reshape(b, t)
    quantize = quant.reshape(b, t, d)
    embed_onehot = onehot.reshape(1, n, c)
    dist_out = dist.reshape(1, b, t, c)
    return (quantize, embed_ind, embed_onehot, dist_out)


# speedup vs baseline: 1.5567x; 1.5525x over previous
import functools

import jax
import jax.numpy as jnp
from jax import lax
from jax.experimental import pallas as pl
from jax.experimental.pallas import tpu as pltpu
from jax.experimental.pallas import tpu_sc as plsc

NUM_CODES = 8192
CODE_DIM = 32
TILE_N = 128


def _vq_tile_kernel(x_ref, embed_ref, dist_ref, onehot_ref, ind_ref, e2_ref):
    i = pl.program_id(0)
    emb = embed_ref[...]

    @pl.when(i == 0)
    def _():
        e2_ref[...] = jnp.sum(emb * emb, axis=-1)[None, :]

    x = x_ref[...]
    x2 = jnp.sum(x * x, axis=-1, keepdims=True)
    xe = jax.lax.dot_general(
        x, emb, (((1,), (1,)), ((), ())),
        preferred_element_type=jnp.float32)
    dist = -(x2 - 2.0 * xe + e2_ref[...])
    dist_ref[...] = dist
    ind = jnp.argmax(dist, axis=-1)
    iota = jax.lax.broadcasted_iota(jnp.int32, dist.shape, 1)
    onehot_ref[...] = (iota == ind[:, None]).astype(jnp.float32)
    ind_ref[0, 0, :] = ind


def _make_sc_gather(n, d):
    info = plsc.get_sparse_core_info()
    nw = info.num_cores * info.num_subcores
    b_per_w = n // nw
    mesh = plsc.VectorSubcoreMesh(core_axis_name="c", subcore_axis_name="s")

    @functools.partial(
        pl.kernel, mesh=mesh,
        out_type=jax.ShapeDtypeStruct((n, d), jnp.float32),
        scratch_types=[
            pltpu.VMEM((b_per_w,), jnp.int32),
            pltpu.VMEM((b_per_w, d), jnp.float32),
            pltpu.SemaphoreType.DMA,
        ],
        compiler_params=pltpu.CompilerParams(use_tc_tiling_on_sc=False),
    )
    def gather_rows(table_hbm, idx_hbm, out_hbm, idx_v, rows_v, sem):
        wid = lax.axis_index("s") * info.num_cores + lax.axis_index("c")
        base = wid * b_per_w
        pltpu.sync_copy(idx_hbm.at[pl.ds(base, b_per_w)], idx_v)
        pltpu.async_copy(table_hbm.at[idx_v], rows_v, sem).wait()
        pltpu.sync_copy(rows_v, out_hbm.at[pl.ds(base, b_per_w)])

    return gather_rows


def kernel(x, embed):
    x = x.astype(jnp.float32)
    b, t, d = x.shape
    n = b * t
    c = embed.shape[1]
    n_tiles = n // TILE_N
    xf = x.reshape(n, d)
    emb = embed.reshape(c, d)

    dist, onehot, ind = pl.pallas_call(
        _vq_tile_kernel,
        grid=(n_tiles,),
        in_specs=[
            pl.BlockSpec((TILE_N, d), lambda i: (i, 0)),
            pl.BlockSpec((c, d), lambda i: (0, 0)),
        ],
        out_specs=[
            pl.BlockSpec((TILE_N, c), lambda i: (i, 0)),
            pl.BlockSpec((TILE_N, c), lambda i: (i, 0)),
            pl.BlockSpec((1, 1, TILE_N), lambda i: (i, 0, 0)),
        ],
        out_shape=[
            jax.ShapeDtypeStruct((n, c), jnp.float32),
            jax.ShapeDtypeStruct((n, c), jnp.float32),
            jax.ShapeDtypeStruct((n_tiles, 1, TILE_N), jnp.int32),
        ],
        scratch_shapes=[pltpu.VMEM((1, c), jnp.float32)],
        compiler_params=pltpu.CompilerParams(
            dimension_semantics=("arbitrary",)),
    )(xf, emb)

    ind_flat = ind.reshape(n)
    quant = _make_sc_gather(n, d)(emb, ind_flat)

    embed_ind = ind_flat.reshape(b, t)
    quantize = quant.reshape(b, t, d)
    embed_onehot = onehot.reshape(1, n, c)
    dist_out = dist.reshape(1, b, t, c)
    return (quantize, embed_ind, embed_onehot, dist_out)
